# baseline (speedup 1.0000x reference)
import jax
import jax.numpy as jnp
from jax import lax
from jax.experimental import pallas as pl
from jax.experimental.pallas import tpu as pltpu
from jax.experimental.pallas import tpu_sc as plsc

D = 64
B = 16384

NC = 2
NS = 16
NW = NC * NS
SPW = B // NW
CHUNK = 128
NCHUNK = SPW // CHUNK


def _score_kernel(hidx_hbm, ridx_hbm, tidx_hbm, ent_hbm, r1_hbm, r2_hbm,
                  out_hbm,
                  hidx_v, ridx_v, tidx_v, h_v, t_v, r1_v, r2_v, p_v, sc_v,
                  sem):
    wid = lax.axis_index("s") * NC + lax.axis_index("c")
    base = wid * SPW
    lane = lax.iota(jnp.int32, 16)

    def chunk_body(ci, _):
        off = base + ci * CHUNK
        pltpu.sync_copy(hidx_hbm.at[pl.ds(off, CHUNK)], hidx_v)
        pltpu.sync_copy(ridx_hbm.at[pl.ds(off, CHUNK)], ridx_v)
        pltpu.sync_copy(tidx_hbm.at[pl.ds(off, CHUNK)], tidx_v)
        cp1 = pltpu.async_copy(ent_hbm.at[hidx_v], h_v, sem)
        cp2 = pltpu.async_copy(ent_hbm.at[tidx_v], t_v, sem)
        cp3 = pltpu.async_copy(r1_hbm.at[ridx_v], r1_v, sem)
        cp4 = pltpu.async_copy(r2_hbm.at[ridx_v], r2_v, sem)
        cp1.wait()
        cp2.wait()
        cp3.wait()
        cp4.wait()

        def group_body(g, _):
            s0 = g * 16
            tot = jnp.zeros((16,), jnp.float32)
            for j in range(16):
                s = s0 + j
                acc = None
                for k in range(D // 16):
                    sl = pl.ds(k * 16, 16)
                    rv = r1_v[s, sl] + r2_v[s, sl]
                    term = h_v[s, sl] * t_v[s, sl] * rv
                    acc = term if acc is None else acc + term
                tot = jnp.where(lane == j, jnp.sum(acc), tot)
            sc_v[pl.ds(ci * CHUNK + s0, 16)] = tot
            return 0

        lax.fori_loop(0, CHUNK // 16, group_body, 0)
        return 0

    lax.fori_loop(0, NCHUNK, chunk_body, 0)
    pltpu.sync_copy(sc_v, out_hbm.at[pl.ds(base, SPW)])


@jax.jit
def _score(hidx, ridx, tidx, ent_emb, rel1, rel2):
    mesh = plsc.VectorSubcoreMesh(core_axis_name="c", subcore_axis_name="s")
    return pl.kernel(
        _score_kernel,
        out_type=jax.ShapeDtypeStruct((B,), jnp.float32),
        mesh=mesh,
        compiler_params=pltpu.CompilerParams(
            needs_layout_passes=False, use_tc_tiling_on_sc=False),
        scratch_types=[
            pltpu.VMEM((CHUNK,), jnp.int32),
            pltpu.VMEM((CHUNK,), jnp.int32),
            pltpu.VMEM((CHUNK,), jnp.int32),
            pltpu.VMEM((CHUNK, D), jnp.float32),
            pltpu.VMEM((CHUNK, D), jnp.float32),
            pltpu.VMEM((CHUNK, D), jnp.float32),
            pltpu.VMEM((CHUNK, D), jnp.float32),
            pltpu.VMEM((256,), jnp.float32),
            pltpu.VMEM((SPW,), jnp.float32),
            pltpu.SemaphoreType.DMA,
        ],
    )(hidx, ridx, tidx, ent_emb, rel1, rel2)


def kernel(sample, ent_emb, relation_embedding, relation_embedding_2):
    sample = sample.astype(jnp.int32)
    hidx = sample[:, 0]
    ridx = sample[:, 1]
    tidx = sample[:, 2]
    scores = _score(hidx, ridx, tidx, ent_emb,
                    relation_embedding, relation_embedding_2)
    return scores[:, None]
